# baseline (speedup 1.0000x reference)
import functools

import jax
import jax.numpy as jnp
from jax import lax
from jax.experimental import pallas as pl
from jax.experimental.pallas import tpu as pltpu
from jax.experimental.pallas import tpu_sc as plsc

NW = 32
D = 512
NV = D // 16

F0, C0 = 25, 2
F1, C1 = 10, 4
CFP0 = 56
CFP1 = 40
CF = max(CFP0, CFP1)


def _sc_gather_sums(embedding, idx_self, idx0, idx1, B):
    bpw = B // NW
    nch0 = bpw // C0
    nch1 = bpw // C1
    mesh = plsc.VectorSubcoreMesh(core_axis_name="c", subcore_axis_name="s")

    @functools.partial(
        pl.kernel,
        mesh=mesh,
        out_type=(
            jax.ShapeDtypeStruct((B, D), jnp.float32),
            jax.ShapeDtypeStruct((B, D), jnp.float32),
            jax.ShapeDtypeStruct((B, D), jnp.float32),
        ),
        scratch_types=[
            pltpu.VMEM((bpw,), jnp.int32),
            pltpu.VMEM((nch0, CFP0), jnp.int32),
            pltpu.VMEM((nch1, CFP1), jnp.int32),
            pltpu.VMEM((CF, D), jnp.float32),
            pltpu.VMEM((CF, D), jnp.float32),
            pltpu.VMEM((bpw // 2, D), jnp.float32),
            pltpu.SemaphoreType.DMA,
            pltpu.SemaphoreType.DMA,
            pltpu.SemaphoreType.DMA,
        ],
    )
    def k(emb_hbm, idxs_hbm, idx0_hbm, idx1_hbm,
          self_hbm, s0_hbm, s1_hbm,
          idxs_v, idx0_v, idx1_v, buf0_v, buf1_v, out_v, sem0, sem1, semS):
        wid = lax.axis_index("s") * 2 + lax.axis_index("c")
        half = bpw // 2
        base = wid * bpw
        bufs = (buf0_v, buf1_v)
        sems = (sem0, sem1)

        pltpu.sync_copy(idxs_hbm.at[wid], idxs_v)
        pltpu.sync_copy(idx0_hbm.at[wid], idx0_v)

        def fire_self(h):
            return pltpu.async_copy(
                emb_hbm.at[idxs_v.at[pl.ds(h * half, half)]], out_v, semS)

        def neigh_half(idx_v, ch0, nch_h, C, F, cf, dst_hbm, dst_off,
                       prologue):
            def fire(ci, b):
                pltpu.async_copy(
                    emb_hbm.at[idx_v.at[ch0 + ci]], bufs[b].at[pl.ds(0, cf)],
                    sems[b])

            def drain(b):
                pltpu.make_async_copy(
                    emb_hbm.at[pl.ds(0, cf)], bufs[b].at[pl.ds(0, cf)],
                    sems[b]).wait()

            def accum(b, ci):
                def node_body(n, _):
                    def row_body(j, accs):
                        return tuple(
                            accs[d] + bufs[b][n * F + j, pl.ds(d * 16, 16)]
                            for d in range(NV)
                        )
                    accs = tuple(
                        bufs[b][n * F, pl.ds(d * 16, 16)] for d in range(NV)
                    )
                    accs = lax.fori_loop(1, F, row_body, accs)
                    for d in range(NV):
                        out_v[ci * C + n, pl.ds(d * 16, 16)] = accs[d]
                    return 0

                lax.fori_loop(0, C, node_body, 0)

            fire(0, 0)
            if prologue is not None:
                prologue()
            fire(1, 1)

            def pair_body(p, _):
                ci0 = 2 * p
                drain(0)
                accum(0, ci0)

                @pl.when(ci0 + 2 < nch_h)
                def _():
                    fire(ci0 + 2, 0)

                drain(1)
                accum(1, ci0 + 1)

                @pl.when(ci0 + 3 < nch_h)
                def _():
                    fire(ci0 + 3, 1)

                return 0

            lax.fori_loop(0, nch_h // 2, pair_body, 0)
            pltpu.sync_copy(out_v, dst_hbm.at[pl.ds(base + dst_off, half)])

        self_cp = fire_self(0)

        def make_self_prologue(h):
            def prologue():
                self_cp.wait()
                pltpu.sync_copy(
                    out_v, self_hbm.at[pl.ds(base + h * half, half)])
            return prologue

        nch0h, nch1h = nch0 // 2, nch1 // 2
        neigh_half(idx0_v, ch0=0, nch_h=nch0h, C=C0, F=F0, cf=CFP0,
                   dst_hbm=s0_hbm, dst_off=0, prologue=make_self_prologue(0))
        self_cp = fire_self(1)
        neigh_half(idx0_v, ch0=nch0h, nch_h=nch0h, C=C0, F=F0, cf=CFP0,
                   dst_hbm=s0_hbm, dst_off=half,
                   prologue=make_self_prologue(1))
        pltpu.sync_copy(idx1_hbm.at[wid], idx1_v)
        neigh_half(idx1_v, ch0=0, nch_h=nch1h, C=C1, F=F1, cf=CFP1,
                   dst_hbm=s1_hbm, dst_off=0, prologue=None)
        neigh_half(idx1_v, ch0=nch1h, nch_h=nch1h, C=C1, F=F1, cf=CFP1,
                   dst_hbm=s1_hbm, dst_off=half, prologue=None)

    return k(embedding, idx_self, idx0, idx1)


def _tc_dense_chain(sv, s0, s1, W_self0, W_neigh0, W_self1, W_neigh1, dense,
                    inv0, inv1):
    B = sv.shape[0]
    BM = 512
    H = W_self0.shape[1]
    L = dense.shape[1]

    def body(sv_ref, s0_ref, s1_ref, ws0, wn0, ws1, wn1, dn, out_ref):
        f32 = jnp.float32
        h = jnp.dot(sv_ref[...], ws0[...], preferred_element_type=f32)
        h += jnp.dot(s0_ref[...] * inv0, wn0[...], preferred_element_type=f32)
        h = jnp.maximum(h, 0.0)
        h2 = jnp.dot(h, ws1[...], preferred_element_type=f32)
        h2 += jnp.dot(s1_ref[...] * inv1, wn1[...], preferred_element_type=f32)
        h2 = jnp.maximum(h2, 0.0)
        out_ref[...] = jax.nn.sigmoid(
            jnp.dot(h2, dn[...], preferred_element_type=f32))

    grid = (B // BM,)
    row_spec = pl.BlockSpec((BM, D), lambda i: (i, 0))
    return pl.pallas_call(
        body,
        grid=grid,
        in_specs=[
            row_spec, row_spec, row_spec,
            pl.BlockSpec((D, H), lambda i: (0, 0)),
            pl.BlockSpec((D, H), lambda i: (0, 0)),
            pl.BlockSpec((H, H), lambda i: (0, 0)),
            pl.BlockSpec((D, H), lambda i: (0, 0)),
            pl.BlockSpec((H, L), lambda i: (0, 0)),
        ],
        out_specs=pl.BlockSpec((BM, L), lambda i: (i, 0)),
        out_shape=jax.ShapeDtypeStruct((B, L), jnp.float32),
    )(sv, s0, s1, W_self0, W_neigh0, W_self1, W_neigh1, dense)


def kernel(nodes, neigh0, neigh1, embedding, W_self0, W_neigh0, W_self1,
           W_neigh1, dense):
    B = nodes.shape[0]
    bpw = B // NW
    idx_self = nodes.astype(jnp.int32).reshape(NW, bpw)
    idx0 = neigh0.astype(jnp.int32).reshape(NW, bpw // C0, C0 * F0)
    idx0 = jnp.pad(idx0, ((0, 0), (0, 0), (0, CFP0 - C0 * F0)))
    idx1 = neigh1.astype(jnp.int32).reshape(NW, bpw // C1, C1 * F1)
    sv, s0, s1 = _sc_gather_sums(embedding, idx_self, idx0, idx1, B)
    return _tc_dense_chain(sv, s0, s1, W_self0, W_neigh0, W_self1, W_neigh1,
                           dense, 1.0 / F0, 1.0 / F1)
